# Initial kernel scaffold; baseline (speedup 1.0000x reference)
import jax
import jax.numpy as jnp
from jax.experimental import pallas as pl


def kernel(x, adj, Wl0, Wr0, bias0, Wl1, Wr1, bias1, Wl2, Wr2, bias2, bn_g0, bn_b0, bn_g1, bn_b1, reg_W1, reg_b1, reg_W2, reg_b2):
    raise NotImplementedError("write your pallas kernel here")


# speedup vs baseline: 1.4389x; 1.4389x over previous
import jax
import jax.numpy as jnp
from jax.experimental import pallas as pl

_N = 10000
_H = 128
_BM = 400
_EPS = 1e-5


def _bf(a):
    return a.astype(jnp.bfloat16)


def _f32dot(a, b):
    return jnp.dot(_bf(a), _bf(b), preferred_element_type=jnp.float32)


def _prep_body(x_ref, wl_ref, wr_ref, b_ref, p_ref, r_ref):
    x = x_ref[...]
    p_ref[...] = _bf(_f32dot(x, wl_ref[...]))
    r_ref[...] = _f32dot(x, wr_ref[...]) + b_ref[...]


def _spmm_cs_body(adj_ref, p_ref, r_ref, z_ref, s_ref, ss_ref, cs_ref):
    @pl.when(pl.program_id(0) == 0)
    def _():
        s_ref[...] = jnp.zeros_like(s_ref)
        ss_ref[...] = jnp.zeros_like(ss_ref)
        cs_ref[...] = jnp.zeros_like(cs_ref)

    a = adj_ref[...]
    z = jnp.dot(_bf(a), p_ref[...], preferred_element_type=jnp.float32) + r_ref[...]
    z_ref[...] = z
    s_ref[...] += jnp.sum(z, axis=0, keepdims=True)
    ss_ref[...] += jnp.sum(z * z, axis=0, keepdims=True)
    cs_ref[...] += jnp.sum(a, axis=0, keepdims=True)


def _spmm_body(adj_ref, p_ref, r_ref, z_ref, s_ref, ss_ref):
    @pl.when(pl.program_id(0) == 0)
    def _():
        s_ref[...] = jnp.zeros_like(s_ref)
        ss_ref[...] = jnp.zeros_like(ss_ref)

    a = adj_ref[...]
    z = jnp.dot(_bf(a), p_ref[...], preferred_element_type=jnp.float32) + r_ref[...]
    z_ref[...] = z
    s_ref[...] += jnp.sum(z, axis=0, keepdims=True)
    ss_ref[...] += jnp.sum(z * z, axis=0, keepdims=True)


def _bn_relu(z, s, ss, g, b):
    n = jnp.float32(_N)
    m = s / n
    v = ss / n - m * m
    return jnp.maximum((z - m) * jax.lax.rsqrt(v + _EPS) * g + b, 0.0)


def _mid_body(z_ref, s_ref, ss_ref, g_ref, b_ref, wl_ref, wr_ref, bias_ref,
              p_ref, r_ref):
    h = _bn_relu(z_ref[...], s_ref[...], ss_ref[...], g_ref[...], b_ref[...])
    p_ref[...] = _bf(_f32dot(h, wl_ref[...]))
    r_ref[...] = _f32dot(h, wr_ref[...]) + bias_ref[...]


def _final_body(z_ref, s_ref, ss_ref, c_ref, g_ref, b_ref, wl_ref, wr_ref,
                bias_ref, w1_ref, b1_ref, w2_ref, b2_ref, y_ref):
    n = jnp.float32(_N)
    h = _bn_relu(z_ref[...], s_ref[...], ss_ref[...], g_ref[...], b_ref[...])
    q = _bf(_f32dot(h, wl_ref[...])).astype(jnp.float32)
    u = jnp.sum(c_ref[...] * q, axis=0, keepdims=True) / n
    hm = jnp.sum(h, axis=0, keepdims=True) / n
    g = u + _f32dot(hm, wr_ref[...]) + bias_ref[...]
    t = jnp.maximum(_f32dot(g, w1_ref[...]) + b1_ref[...], 0.0)
    y_ref[...] = _f32dot(t, w2_ref[...]) + b2_ref[...]


def _spmm_pass(adj, p, r, with_colsum):
    f32 = jnp.float32
    out_shape = [
        jax.ShapeDtypeStruct((_N, _H), f32),
        jax.ShapeDtypeStruct((1, _H), f32),
        jax.ShapeDtypeStruct((1, _H), f32),
    ]
    out_specs = [
        pl.BlockSpec((_BM, _H), lambda i: (i, 0)),
        pl.BlockSpec((1, _H), lambda i: (0, 0)),
        pl.BlockSpec((1, _H), lambda i: (0, 0)),
    ]
    body = _spmm_body
    if with_colsum:
        out_shape.append(jax.ShapeDtypeStruct((1, _N), f32))
        out_specs.append(pl.BlockSpec((1, _N), lambda i: (0, 0)))
        body = _spmm_cs_body
    return pl.pallas_call(
        body,
        grid=(_N // _BM,),
        in_specs=[
            pl.BlockSpec((_BM, _N), lambda i: (i, 0)),
            pl.BlockSpec((_N, _H), lambda i: (0, 0)),
            pl.BlockSpec((_BM, _H), lambda i: (i, 0)),
        ],
        out_specs=out_specs,
        out_shape=out_shape,
    )(adj, p, r)


def kernel(x, adj, Wl0, Wr0, bias0, Wl1, Wr1, bias1, Wl2, Wr2, bias2,
           bn_g0, bn_b0, bn_g1, bn_b1, reg_W1, reg_b1, reg_W2, reg_b2):
    f32 = jnp.float32
    b0 = bias0.reshape(1, _H)
    b1 = bias1.reshape(1, _H)
    b2 = bias2.reshape(1, _H)
    g0 = bn_g0.reshape(1, _H)
    bb0 = bn_b0.reshape(1, _H)
    g1 = bn_g1.reshape(1, _H)
    bb1 = bn_b1.reshape(1, _H)
    rb1 = reg_b1.reshape(1, _H)
    rb2 = reg_b2.reshape(1, 1)

    p0, r0 = pl.pallas_call(
        _prep_body,
        out_shape=[jax.ShapeDtypeStruct((_N, _H), jnp.bfloat16),
                   jax.ShapeDtypeStruct((_N, _H), f32)],
    )(x, Wl0, Wr0, b0)

    z1, s1, ss1, cs = _spmm_pass(adj, p0, r0, with_colsum=True)

    p1, r1 = pl.pallas_call(
        _mid_body,
        out_shape=[jax.ShapeDtypeStruct((_N, _H), jnp.bfloat16),
                   jax.ShapeDtypeStruct((_N, _H), f32)],
    )(z1, s1, ss1, g0, bb0, Wl1, Wr1, b1)

    z2, s2, ss2 = _spmm_pass(adj, p1, r1, with_colsum=False)

    c = cs.reshape(_N, 1)
    y = pl.pallas_call(
        _final_body,
        out_shape=jax.ShapeDtypeStruct((1, 1), f32),
    )(z2, s2, ss2, c, g1, bb1, Wl2, Wr2, b2, reg_W1, rb1, reg_W2, rb2)
    return y.reshape(())
